# baseline (speedup 1.0000x reference)
import functools

import jax
import jax.numpy as jnp
from jax import lax
from jax.experimental import pallas as pl
from jax.experimental.pallas import tpu as pltpu
from jax.experimental.pallas import tpu_sc as plsc

NC = 2
NS = 16
NW = NC * NS

EC = 2000
GCH = 80
PCH = 128
DEGW = 16


def _seg_sum_kernel(Np, D, E):
  W = D + DEGW
  rpw = Np // NW
  nchunk = E // EC
  ngrp = EC // 16
  mesh = plsc.VectorSubcoreMesh(
      core_axis_name="c", subcore_axis_name="s", num_cores=NC,
      num_subcores=NS)

  @functools.partial(
      pl.kernel,
      out_type=jax.ShapeDtypeStruct((Np, W), jnp.float32),
      mesh=mesh,
      compiler_params=pltpu.CompilerParams(needs_layout_passes=False),
      scratch_types=[
          pltpu.VMEM((rpw, W), jnp.float32),
          pltpu.VMEM((EC,), jnp.int32),
          pltpu.VMEM((EC,), jnp.int32),
          pltpu.VMEM((EC,), jnp.int32),
          pltpu.VMEM((EC,), jnp.int32),
          pltpu.VMEM((EC + GCH,), jnp.int32),
          pltpu.VMEM((EC + GCH,), jnp.int32),
          pltpu.VMEM((EC + GCH,), jnp.int32),
          pltpu.VMEM((EC + GCH,), jnp.int32),
          pltpu.VMEM((GCH, D), jnp.float32),
          pltpu.VMEM((GCH, D), jnp.float32),
          pltpu.SemaphoreType.DMA,
          pltpu.SemaphoreType.DMA,
          pltpu.SemaphoreType.DMA,
          pltpu.SemaphoreType.DMA,
          pltpu.SemaphoreType.DMA,
          pltpu.SemaphoreType.DMA,
      ],
  )
  def k(x_hbm, esrc_hbm, edst_hbm, out_hbm,
        acc_v, sbufa_v, dbufa_v, sbufb_v, dbufb_v,
        selsa_v, selda_v, selsb_v, seldb_v, rowsa_v, rowsb_v,
        ssa, sda, ssb, sdb, sga, sgb):
    c = lax.axis_index("c")
    s = lax.axis_index("s")
    wid = s * NC + c
    lo = wid * rpw
    iota = lax.iota(jnp.int32, 16)
    zeros16 = jnp.zeros((16,), jnp.float32)
    ones16 = jnp.ones((16,), jnp.float32)

    @pl.loop(0, rpw)
    def _zr(r):
      @pl.loop(0, W // 16)
      def _zc(cb):
        acc_v[r, pl.ds(cb * 16, 16)] = zeros16

    @pl.loop(0, (EC + GCH) // 16)
    def _zs(r):
      selsa_v[pl.ds(r * 16, 16)] = jnp.zeros((16,), jnp.int32)
      selsb_v[pl.ds(r * 16, 16)] = jnp.zeros((16,), jnp.int32)
      selda_v[pl.ds(r * 16, 16)] = jnp.zeros((16,), jnp.int32)
      seldb_v[pl.ds(r * 16, 16)] = jnp.zeros((16,), jnp.int32)

    def scan_chunk(sbuf_v, dbuf_v, selsrc_v, seld_v):
      def _scan(g, csr_vec):
        d16 = dbuf_v[pl.ds(g * 16, 16)]
        s16 = sbuf_v[pl.ds(g * 16, 16)]
        dl = d16 - lo
        m = (dl >= 0) & (dl < rpw)
        mi = jnp.where(m, 1, 0).astype(jnp.int32)
        csum = plsc.cumsum(mi)
        pos = csr_vec + csum - 1
        plsc.store_scatter(seld_v, [pos], dl, mask=m)
        plsc.store_scatter(selsrc_v, [pos], s16, mask=m)
        return csr_vec + plsc.all_reduce_population_count(m)

      csr_vec = pl.loop(0, ngrp, init_carry=jnp.zeros((16,), jnp.int32),
                        unroll=2)(_scan)
      return lax.reduce_max(csr_vec, axes=(0,))

    def accum_window(seld_v, rows_v, fbase, cnt):
      ec = jnp.minimum(cnt - fbase, GCH)

      @pl.loop(0, ec)
      def _edge(e):
        dlb = plsc.load_gather(seld_v, [jnp.full((16,), fbase + e,
                                                 jnp.int32)])
        for cb in range(D // 16):
          v = rows_v[e, pl.ds(cb * 16, 16)]
          plsc.addupdate_scatter(acc_v, [dlb, cb * 16 + iota], v)
        plsc.addupdate_scatter(acc_v, [dlb, D + iota], ones16)

    def accum_window0_static(seld_v, rows_v, cnt):
      for e in range(GCH):
        mask = jnp.full((16,), e < cnt)
        dlb = plsc.load_gather(seld_v, [jnp.full((16,), e, jnp.int32)])
        for cb in range(D // 16):
          v = rows_v[e, pl.ds(cb * 16, 16)]
          plsc.addupdate_scatter(acc_v, [dlb, cb * 16 + iota], v, mask=mask)
        plsc.addupdate_scatter(acc_v, [dlb, D + iota], ones16, mask=mask)

    def accum_rest(selsrc_v, seld_v, rows_v, sem, cnt):
      nf = (cnt + (GCH - 1)) // GCH

      @pl.loop(1, nf)
      def _flush(f):
        fbase = f * GCH
        pltpu.async_copy(x_hbm.at[selsrc_v.at[pl.ds(fbase, GCH)]], rows_v,
                         sem).wait()
        accum_window(seld_v, rows_v, fbase, cnt)

    npair = nchunk // 2
    pltpu.async_copy(esrc_hbm.at[pl.ds(0, EC)], sbufa_v, ssa)
    pltpu.async_copy(edst_hbm.at[pl.ds(0, EC)], dbufa_v, sda)

    @pl.loop(0, npair)
    def _pair(kp):
      a = 2 * kp
      b = a + 1
      nb = jnp.minimum(a + 2, nchunk - 2) * EC

      pltpu.make_async_copy(esrc_hbm.at[pl.ds(0, EC)], sbufa_v, ssa).wait()
      pltpu.make_async_copy(edst_hbm.at[pl.ds(0, EC)], dbufa_v, sda).wait()
      pltpu.async_copy(esrc_hbm.at[pl.ds(b * EC, EC)], sbufb_v, ssb)
      pltpu.async_copy(edst_hbm.at[pl.ds(b * EC, EC)], dbufb_v, sdb)

      cnta = scan_chunk(sbufa_v, dbufa_v, selsa_v, selda_v)
      ga = pltpu.async_copy(x_hbm.at[selsa_v.at[pl.ds(0, GCH)]], rowsa_v,
                            sga)

      pltpu.make_async_copy(esrc_hbm.at[pl.ds(0, EC)], sbufb_v, ssb).wait()
      pltpu.make_async_copy(edst_hbm.at[pl.ds(0, EC)], dbufb_v, sdb).wait()
      pltpu.async_copy(esrc_hbm.at[pl.ds(nb, EC)], sbufa_v, ssa)
      pltpu.async_copy(edst_hbm.at[pl.ds(nb, EC)], dbufa_v, sda)

      cntb = scan_chunk(sbufb_v, dbufb_v, selsb_v, seldb_v)
      ga.wait()
      gb = pltpu.async_copy(x_hbm.at[selsb_v.at[pl.ds(0, GCH)]], rowsb_v,
                            sgb)
      accum_window0_static(selda_v, rowsa_v, cnta)
      accum_rest(selsa_v, selda_v, rowsa_v, sga, cnta)
      gb.wait()
      accum_window0_static(seldb_v, rowsb_v, cntb)
      accum_rest(selsb_v, seldb_v, rowsb_v, sgb, cntb)

    pltpu.make_async_copy(esrc_hbm.at[pl.ds(0, EC)], sbufa_v, ssa).wait()
    pltpu.make_async_copy(edst_hbm.at[pl.ds(0, EC)], dbufa_v, sda).wait()

    pltpu.sync_copy(acc_v, out_hbm.at[pl.ds(lo, rpw)])

  return k


def _gcn_tc_kernel(N, D, bs):
  grid = (N + bs - 1) // bs
  W = D + DEGW

  def body(x_ref, agg_ref, ws_ref, wn_ref, h_ref):
    agg = agg_ref[:, :D]
    deg = agg_ref[:, D:D + 1]
    mean = agg / jnp.maximum(deg, 1.0)
    h = (jnp.dot(x_ref[...], ws_ref[...], preferred_element_type=jnp.float32)
         + jnp.dot(mean, wn_ref[...], preferred_element_type=jnp.float32))
    h_ref[...] = jnp.maximum(h, 0.0)

  return pl.pallas_call(
      body,
      out_shape=jax.ShapeDtypeStruct((N, D), jnp.float32),
      grid=(grid,),
      in_specs=[
          pl.BlockSpec((bs, D), lambda i: (i, 0)),
          pl.BlockSpec((bs, W), lambda i: (i, 0)),
          pl.BlockSpec((D, D), lambda i: (0, 0)),
          pl.BlockSpec((D, D), lambda i: (0, 0)),
      ],
      out_specs=pl.BlockSpec((bs, D), lambda i: (i, 0)),
  )


def _score_kernel(N, D, B, NB):
  ppw = B // NW
  nchunk = ppw // PCH
  ngrp = PCH // 16
  mesh = plsc.VectorSubcoreMesh(
      core_axis_name="c", subcore_axis_name="s", num_cores=NC,
      num_subcores=NS)

  @functools.partial(
      pl.kernel,
      out_type=jax.ShapeDtypeStruct((B,), jnp.float32),
      mesh=mesh,
      compiler_params=pltpu.CompilerParams(needs_layout_passes=False),
      scratch_types=[
          pltpu.VMEM((PCH, D), jnp.float32),
          pltpu.VMEM((PCH, D), jnp.float32),
          pltpu.VMEM((PCH,), jnp.int32),
          pltpu.VMEM((PCH,), jnp.int32),
          pltpu.VMEM((NB,), jnp.float32),
          pltpu.VMEM((B // NW,), jnp.float32),
          pltpu.SemaphoreType.DMA,
          pltpu.SemaphoreType.DMA,
      ],
  )
  def k(h_hbm, src_hbm, dst_hbm, bias_hbm, out_hbm,
        hsrc_v, hdst_v, sidx_v, didx_v, bias_v, out_v, sem1, sem2):
    c = lax.axis_index("c")
    s = lax.axis_index("s")
    wid = s * NC + c
    pltpu.sync_copy(bias_hbm, bias_v)
    base0 = wid * ppw
    iota = lax.iota(jnp.int32, 16)

    @pl.loop(0, nchunk)
    def _chunk(kc):
      base = base0 + kc * PCH
      pltpu.sync_copy(src_hbm.at[pl.ds(base, PCH)], sidx_v)
      pltpu.sync_copy(dst_hbm.at[pl.ds(base, PCH)], didx_v)
      d1 = pltpu.async_copy(h_hbm.at[sidx_v], hsrc_v, sem1)
      d2 = pltpu.async_copy(h_hbm.at[didx_v], hdst_v, sem2)
      d1.wait()
      d2.wait()

      @pl.loop(0, ngrp)
      def _grp(g):
        rows = g * 16 + iota
        si = sidx_v[pl.ds(g * 16, 16)]
        di = didx_v[pl.ds(g * 16, 16)]
        bsum = (plsc.load_gather(bias_v, [si + 1])
                + plsc.load_gather(bias_v, [di + 1]))

        @pl.loop(0, D, init_carry=bsum, unroll=8)
        def _feat(d, acc):
          col = jnp.full((16,), d, jnp.int32)
          a = plsc.load_gather(hsrc_v, [rows, col])
          b = plsc.load_gather(hdst_v, [rows, col])
          return acc + a * b

        out_v[pl.ds(kc * PCH + g * 16, 16)] = _feat

    pltpu.sync_copy(out_v, out_hbm.at[pl.ds(base0, ppw)])

  return k


def kernel(x, edge_index, src, dst, W_self, W_neigh, node_biases):
  N, D = x.shape
  E = edge_index.shape[1]
  B = src.shape[0]

  e_src = edge_index[0]
  e_dst = edge_index[1]
  Np = ((N + 8 * NW - 1) // (8 * NW)) * (8 * NW)
  NB = ((N + 1 + 7) // 8) * 8
  bias_p = jnp.pad(node_biases, (0, NB - (N + 1)))

  aggdeg = _seg_sum_kernel(Np, D, E)(x, e_src, e_dst)

  h = _gcn_tc_kernel(N, D, 512)(x, aggdeg, W_self, W_neigh)

  score = _score_kernel(N, D, B, NB)(h, src, dst, bias_p)
  return score


# speedup vs baseline: 1.0659x; 1.0659x over previous
import functools

import jax
import jax.numpy as jnp
from jax import lax
from jax.experimental import pallas as pl
from jax.experimental.pallas import tpu as pltpu
from jax.experimental.pallas import tpu_sc as plsc

NC = 2
NS = 16
NW = NC * NS

EC = 2000
GCH = 80
PCH = 128
DEGW = 16


def _seg_sum_kernel(Np, D, E):
  W = D + DEGW
  rpw = Np // NW
  nchunk = E // EC
  ngrp = EC // 16
  mesh = plsc.VectorSubcoreMesh(
      core_axis_name="c", subcore_axis_name="s", num_cores=NC,
      num_subcores=NS)

  @functools.partial(
      pl.kernel,
      out_type=jax.ShapeDtypeStruct((Np, W), jnp.float32),
      mesh=mesh,
      compiler_params=pltpu.CompilerParams(needs_layout_passes=False),
      scratch_types=[
          pltpu.VMEM((rpw, W), jnp.float32),
          pltpu.VMEM((EC,), jnp.int32),
          pltpu.VMEM((EC,), jnp.int32),
          pltpu.VMEM((EC,), jnp.int32),
          pltpu.VMEM((EC,), jnp.int32),
          pltpu.VMEM((EC + GCH,), jnp.int32),
          pltpu.VMEM((EC + GCH,), jnp.int32),
          pltpu.VMEM((EC + GCH,), jnp.int32),
          pltpu.VMEM((EC + GCH,), jnp.int32),
          pltpu.VMEM((GCH, D), jnp.float32),
          pltpu.VMEM((GCH, D), jnp.float32),
          pltpu.SemaphoreType.DMA,
          pltpu.SemaphoreType.DMA,
          pltpu.SemaphoreType.DMA,
          pltpu.SemaphoreType.DMA,
          pltpu.SemaphoreType.DMA,
          pltpu.SemaphoreType.DMA,
      ],
  )
  def k(x_hbm, esrc_hbm, edst_hbm, out_hbm,
        acc_v, sbufa_v, dbufa_v, sbufb_v, dbufb_v,
        selsa_v, selda_v, selsb_v, seldb_v, rowsa_v, rowsb_v,
        ssa, sda, ssb, sdb, sga, sgb):
    c = lax.axis_index("c")
    s = lax.axis_index("s")
    wid = s * NC + c
    lo = wid * rpw
    iota = lax.iota(jnp.int32, 16)
    zeros16 = jnp.zeros((16,), jnp.float32)
    ones16 = jnp.ones((16,), jnp.float32)

    @pl.loop(0, rpw)
    def _zr(r):
      @pl.loop(0, W // 16)
      def _zc(cb):
        acc_v[r, pl.ds(cb * 16, 16)] = zeros16

    @pl.loop(0, (EC + GCH) // 16)
    def _zs(r):
      selsa_v[pl.ds(r * 16, 16)] = jnp.zeros((16,), jnp.int32)
      selsb_v[pl.ds(r * 16, 16)] = jnp.zeros((16,), jnp.int32)
      selda_v[pl.ds(r * 16, 16)] = jnp.zeros((16,), jnp.int32)
      seldb_v[pl.ds(r * 16, 16)] = jnp.zeros((16,), jnp.int32)

    def scan_chunk(sbuf_v, dbuf_v, selsrc_v, seld_v):
      def _scan(g, csr_vec):
        d16 = dbuf_v[pl.ds(g * 16, 16)]
        s16 = sbuf_v[pl.ds(g * 16, 16)]
        dl = d16 - lo
        m = (dl >= 0) & (dl < rpw)
        mi = jnp.where(m, 1, 0).astype(jnp.int32)
        csum = plsc.cumsum(mi)
        pos = csr_vec + csum - 1
        plsc.store_scatter(seld_v, [pos], dl, mask=m)
        plsc.store_scatter(selsrc_v, [pos], s16, mask=m)
        return csr_vec + plsc.all_reduce_population_count(m)

      csr_vec = pl.loop(0, ngrp, init_carry=jnp.zeros((16,), jnp.int32),
                        unroll=2)(_scan)
      return lax.reduce_max(csr_vec, axes=(0,))

    def accum_window(seld_v, rows_v, fbase, cnt):
      ec = jnp.minimum(cnt - fbase, GCH)

      @pl.loop(0, ec)
      def _edge(e):
        dlb = plsc.load_gather(seld_v, [jnp.full((16,), fbase + e,
                                                 jnp.int32)])
        for cb in range(D // 16):
          v = rows_v[e, pl.ds(cb * 16, 16)]
          plsc.addupdate_scatter(acc_v, [dlb, cb * 16 + iota], v)
        plsc.addupdate_scatter(acc_v, [dlb, D + iota], ones16)

    def accum_rest(selsrc_v, seld_v, rows_v, sem, cnt):
      nf = (cnt + (GCH - 1)) // GCH

      @pl.loop(1, nf)
      def _flush(f):
        fbase = f * GCH
        pltpu.async_copy(x_hbm.at[selsrc_v.at[pl.ds(fbase, GCH)]], rows_v,
                         sem).wait()
        accum_window(seld_v, rows_v, fbase, cnt)

    npair = nchunk // 2
    pltpu.async_copy(esrc_hbm.at[pl.ds(0, EC)], sbufa_v, ssa)
    pltpu.async_copy(edst_hbm.at[pl.ds(0, EC)], dbufa_v, sda)

    @pl.loop(0, npair)
    def _pair(kp):
      a = 2 * kp
      b = a + 1
      nb = jnp.minimum(a + 2, nchunk - 2) * EC

      pltpu.make_async_copy(esrc_hbm.at[pl.ds(0, EC)], sbufa_v, ssa).wait()
      pltpu.make_async_copy(edst_hbm.at[pl.ds(0, EC)], dbufa_v, sda).wait()
      pltpu.async_copy(esrc_hbm.at[pl.ds(b * EC, EC)], sbufb_v, ssb)
      pltpu.async_copy(edst_hbm.at[pl.ds(b * EC, EC)], dbufb_v, sdb)

      cnta = scan_chunk(sbufa_v, dbufa_v, selsa_v, selda_v)
      ga = pltpu.async_copy(x_hbm.at[selsa_v.at[pl.ds(0, GCH)]], rowsa_v,
                            sga)

      pltpu.make_async_copy(esrc_hbm.at[pl.ds(0, EC)], sbufb_v, ssb).wait()
      pltpu.make_async_copy(edst_hbm.at[pl.ds(0, EC)], dbufb_v, sdb).wait()
      pltpu.async_copy(esrc_hbm.at[pl.ds(nb, EC)], sbufa_v, ssa)
      pltpu.async_copy(edst_hbm.at[pl.ds(nb, EC)], dbufa_v, sda)

      cntb = scan_chunk(sbufb_v, dbufb_v, selsb_v, seldb_v)
      ga.wait()
      gb = pltpu.async_copy(x_hbm.at[selsb_v.at[pl.ds(0, GCH)]], rowsb_v,
                            sgb)
      accum_window(selda_v, rowsa_v, 0, cnta)
      accum_rest(selsa_v, selda_v, rowsa_v, sga, cnta)
      gb.wait()
      accum_window(seldb_v, rowsb_v, 0, cntb)
      accum_rest(selsb_v, seldb_v, rowsb_v, sgb, cntb)

    pltpu.make_async_copy(esrc_hbm.at[pl.ds(0, EC)], sbufa_v, ssa).wait()
    pltpu.make_async_copy(edst_hbm.at[pl.ds(0, EC)], dbufa_v, sda).wait()

    pltpu.sync_copy(acc_v, out_hbm.at[pl.ds(lo, rpw)])

  return k


def _gcn_tc_kernel(N, D, bs):
  grid = (N + bs - 1) // bs
  W = D + DEGW

  def body(x_ref, agg_ref, ws_ref, wn_ref, h_ref):
    agg = agg_ref[:, :D]
    deg = agg_ref[:, D:D + 1]
    mean = agg / jnp.maximum(deg, 1.0)
    h = (jnp.dot(x_ref[...], ws_ref[...], preferred_element_type=jnp.float32)
         + jnp.dot(mean, wn_ref[...], preferred_element_type=jnp.float32))
    h_ref[...] = jnp.maximum(h, 0.0)

  return pl.pallas_call(
      body,
      out_shape=jax.ShapeDtypeStruct((N, D), jnp.float32),
      grid=(grid,),
      in_specs=[
          pl.BlockSpec((bs, D), lambda i: (i, 0)),
          pl.BlockSpec((bs, W), lambda i: (i, 0)),
          pl.BlockSpec((D, D), lambda i: (0, 0)),
          pl.BlockSpec((D, D), lambda i: (0, 0)),
      ],
      out_specs=pl.BlockSpec((bs, D), lambda i: (i, 0)),
  )


def _score_kernel(N, D, B, NB):
  ppw = B // NW
  nchunk = ppw // PCH
  ngrp = PCH // 16
  mesh = plsc.VectorSubcoreMesh(
      core_axis_name="c", subcore_axis_name="s", num_cores=NC,
      num_subcores=NS)

  @functools.partial(
      pl.kernel,
      out_type=jax.ShapeDtypeStruct((B,), jnp.float32),
      mesh=mesh,
      compiler_params=pltpu.CompilerParams(needs_layout_passes=False),
      scratch_types=[
          pltpu.VMEM((PCH, D), jnp.float32),
          pltpu.VMEM((PCH, D), jnp.float32),
          pltpu.VMEM((PCH,), jnp.int32),
          pltpu.VMEM((PCH,), jnp.int32),
          pltpu.VMEM((NB,), jnp.float32),
          pltpu.VMEM((B // NW,), jnp.float32),
          pltpu.SemaphoreType.DMA,
          pltpu.SemaphoreType.DMA,
      ],
  )
  def k(h_hbm, src_hbm, dst_hbm, bias_hbm, out_hbm,
        hsrc_v, hdst_v, sidx_v, didx_v, bias_v, out_v, sem1, sem2):
    c = lax.axis_index("c")
    s = lax.axis_index("s")
    wid = s * NC + c
    pltpu.sync_copy(bias_hbm, bias_v)
    base0 = wid * ppw
    iota = lax.iota(jnp.int32, 16)

    @pl.loop(0, nchunk)
    def _chunk(kc):
      base = base0 + kc * PCH
      pltpu.sync_copy(src_hbm.at[pl.ds(base, PCH)], sidx_v)
      pltpu.sync_copy(dst_hbm.at[pl.ds(base, PCH)], didx_v)
      d1 = pltpu.async_copy(h_hbm.at[sidx_v], hsrc_v, sem1)
      d2 = pltpu.async_copy(h_hbm.at[didx_v], hdst_v, sem2)
      d1.wait()
      d2.wait()

      @pl.loop(0, ngrp)
      def _grp(g):
        rows = g * 16 + iota
        si = sidx_v[pl.ds(g * 16, 16)]
        di = didx_v[pl.ds(g * 16, 16)]
        bsum = (plsc.load_gather(bias_v, [si + 1])
                + plsc.load_gather(bias_v, [di + 1]))

        @pl.loop(0, D, init_carry=bsum, unroll=8)
        def _feat(d, acc):
          col = jnp.full((16,), d, jnp.int32)
          a = plsc.load_gather(hsrc_v, [rows, col])
          b = plsc.load_gather(hdst_v, [rows, col])
          return acc + a * b

        out_v[pl.ds(kc * PCH + g * 16, 16)] = _feat

    pltpu.sync_copy(out_v, out_hbm.at[pl.ds(base0, ppw)])

  return k


def kernel(x, edge_index, src, dst, W_self, W_neigh, node_biases):
  N, D = x.shape
  E = edge_index.shape[1]
  B = src.shape[0]

  e_src = edge_index[0]
  e_dst = edge_index[1]
  Np = ((N + 8 * NW - 1) // (8 * NW)) * (8 * NW)
  NB = ((N + 1 + 7) // 8) * 8
  bias_p = jnp.pad(node_biases, (0, NB - (N + 1)))

  aggdeg = _seg_sum_kernel(Np, D, E)(x, e_src, e_dst)

  h = _gcn_tc_kernel(N, D, 512)(x, aggdeg, W_self, W_neigh)

  score = _score_kernel(N, D, B, NB)(h, src, dst, bias_p)
  return score
